# baseline (speedup 1.0000x reference)
import functools

import jax
import jax.numpy as jnp
from jax import lax
from jax.experimental import pallas as pl
from jax.experimental.pallas import tpu as pltpu
from jax.experimental.pallas import tpu_sc as plsc


def _make_sc_gather(b, input_dim, d):
    info = plsc.get_sparse_core_info()
    nc, ns = info.num_cores, info.num_subcores
    nw = nc * ns
    assert b % nw == 0
    per_w = (b // nw) * input_dim
    assert per_w % 8 == 0

    mesh = plsc.VectorSubcoreMesh(core_axis_name="c", subcore_axis_name="s")

    @functools.partial(
        pl.kernel,
        mesh=mesh,
        out_type=jax.ShapeDtypeStruct((b * input_dim, d), jnp.float32),
        scratch_types=[
            pltpu.VMEM((per_w,), jnp.int32),
            pltpu.VMEM((per_w, d), jnp.float32),
            pltpu.SemaphoreType.DMA,
        ],
        compiler_params=pltpu.CompilerParams(use_tc_tiling_on_sc=False),
    )
    def gather(table_hbm, idx_hbm, out_hbm, idx_v, rows_v, sem):
        wid = lax.axis_index("s") * nc + lax.axis_index("c")
        base = wid * per_w
        pltpu.sync_copy(idx_hbm.at[pl.ds(base, per_w)], idx_v)
        pltpu.async_copy(table_hbm.at[idx_v], rows_v, sem).wait()
        pltpu.sync_copy(rows_v, out_hbm.at[pl.ds(base, per_w)])

    return gather


def _mlp_body(flat_ref, w1_ref, b1_ref, w2_ref, b2_ref, out_ref):
    h = jnp.dot(flat_ref[...], w1_ref[...], preferred_element_type=jnp.float32)
    h = jnp.maximum(h + b1_ref[...], 0.0)
    out_ref[...] = (
        jnp.dot(h, w2_ref[...], preferred_element_type=jnp.float32) + b2_ref[...]
    )


def _mlp(flat, W1, b1, W2, b2, blk):
    b, mlp_in = flat.shape
    hidden = W1.shape[1]
    out_sz = W2.shape[1]
    return pl.pallas_call(
        _mlp_body,
        grid=(b // blk,),
        in_specs=[
            pl.BlockSpec((blk, mlp_in), lambda i: (i, 0)),
            pl.BlockSpec((mlp_in, hidden), lambda i: (0, 0)),
            pl.BlockSpec((1, hidden), lambda i: (0, 0)),
            pl.BlockSpec((hidden, out_sz), lambda i: (0, 0)),
            pl.BlockSpec((1, out_sz), lambda i: (0, 0)),
        ],
        out_specs=pl.BlockSpec((blk, out_sz), lambda i: (i, 0)),
        out_shape=jax.ShapeDtypeStruct((b, out_sz), jnp.float32),
    )(flat, W1, b1.reshape(1, -1), W2, b2.reshape(1, -1))


def kernel(x, emb_table, W1, b1, W2, b2):
    b, input_dim = x.shape
    vocab, d = emb_table.shape
    n = b * input_dim
    half = vocab // 2
    gather = _make_sc_gather(b, input_dim, d)
    idx = x.reshape(n)
    idx_a = jnp.minimum(idx, half - 1)
    idx_b = jnp.maximum(idx - half, 0)
    flat_a = gather(emb_table[:half], idx_a)
    flat_b = gather(emb_table[half:], idx_b)
    sel = (idx < half)[:, None]
    flat = jnp.where(sel, flat_a, flat_b).reshape(b, input_dim * d)
    return _mlp(flat, W1, b1, W2, b2, blk=512)


# speedup vs baseline: 3.4170x; 3.4170x over previous
import functools

import jax
import jax.numpy as jnp
from jax import lax
from jax.experimental import pallas as pl
from jax.experimental.pallas import tpu as pltpu
from jax.experimental.pallas import tpu_sc as plsc


def _make_sc_gather(b, input_dim, d):
    info = plsc.get_sparse_core_info()
    nc, ns = info.num_cores, info.num_subcores
    nw = nc * ns
    assert b % nw == 0
    per_w = (b // nw) * input_dim
    assert per_w % 8 == 0

    mesh = plsc.VectorSubcoreMesh(core_axis_name="c", subcore_axis_name="s")

    @functools.partial(
        pl.kernel,
        mesh=mesh,
        out_type=jax.ShapeDtypeStruct((b * input_dim, d), jnp.float32),
        scratch_types=[
            pltpu.VMEM((per_w,), jnp.int32),
            pltpu.VMEM((per_w, d), jnp.float32),
            pltpu.SemaphoreType.DMA,
        ],
        compiler_params=pltpu.CompilerParams(use_tc_tiling_on_sc=False),
    )
    def gather(table_hbm, idx_hbm, out_hbm, idx_v, rows_v, sem):
        wid = lax.axis_index("s") * nc + lax.axis_index("c")
        base = wid * per_w
        pltpu.sync_copy(idx_hbm.at[pl.ds(base, per_w)], idx_v)
        pltpu.async_copy(table_hbm.at[idx_v], rows_v, sem).wait()
        pltpu.sync_copy(rows_v, out_hbm.at[pl.ds(base, per_w)])

    return gather


def _mlp_body(flat_ref, w1_ref, b1_ref, w2_ref, b2_ref, out_ref):
    h = jnp.dot(flat_ref[...], w1_ref[...], preferred_element_type=jnp.float32)
    h = jnp.maximum(h + b1_ref[...], 0.0)
    out_ref[...] = (
        jnp.dot(h, w2_ref[...], preferred_element_type=jnp.float32) + b2_ref[...]
    )


def _mlp(flat, W1, b1, W2, b2, blk):
    b, mlp_in = flat.shape
    hidden = W1.shape[1]
    out_sz = W2.shape[1]
    return pl.pallas_call(
        _mlp_body,
        grid=(b // blk,),
        in_specs=[
            pl.BlockSpec((blk, mlp_in), lambda i: (i, 0)),
            pl.BlockSpec((mlp_in, hidden), lambda i: (0, 0)),
            pl.BlockSpec((1, hidden), lambda i: (0, 0)),
            pl.BlockSpec((hidden, out_sz), lambda i: (0, 0)),
            pl.BlockSpec((1, out_sz), lambda i: (0, 0)),
        ],
        out_specs=pl.BlockSpec((blk, out_sz), lambda i: (i, 0)),
        out_shape=jax.ShapeDtypeStruct((b, out_sz), jnp.float32),
    )(flat, W1, b1.reshape(1, -1), W2, b2.reshape(1, -1))


def kernel(x, emb_table, W1, b1, W2, b2):
    b, input_dim = x.shape
    vocab, d = emb_table.shape
    gather = _make_sc_gather(b, input_dim, d)
    flat = gather(emb_table, x.reshape(b * input_dim)).reshape(b, input_dim * d)
    return _mlp(flat, W1, b1, W2, b2, blk=512)
